# baseline (speedup 1.0000x reference)
import functools

import jax
import jax.numpy as jnp
from jax import lax
from jax.experimental import pallas as pl
from jax.experimental.pallas import tpu as pltpu
from jax.experimental.pallas import tpu_sc as plsc

VOCAB = 1000000
DIM = 64
MAXLEN = 200
BATCH = 1024

TOKENS = BATCH * MAXLEN
NW = 32
PER_W = TOKENS // NW
CHUNK = 640
GATHERS = CHUNK // 128
STEPS = PER_W // CHUNK


def _emb_kernel(x_hbm, tok_hbm, pos_hbm, out_hbm, idx_v, rows_v, pos_v, sem):
    c = lax.axis_index("c")
    s = lax.axis_index("s")
    wid = s * 2 + c

    pltpu.sync_copy(pos_hbm, pos_v)

    def step(st, carry):
        base = wid * PER_W + st * CHUNK
        page = wid * STEPS + st
        pltpu.sync_copy(x_hbm.at[page], idx_v)
        hs = []
        for j in range(GATHERS):
            hs.append(pltpu.async_copy(
                tok_hbm.at[idx_v.at[j]],
                rows_v.at[pl.ds(j * 128, 128)],
                sem,
            ))
        for h in hs:
            h.wait()

        off = lax.rem(st * CHUNK, MAXLEN)

        def add_pos(t, _):
            p = lax.rem(off + t, MAXLEN)
            for d in range(DIM // 16):
                sl = pl.ds(d * 16, 16)
                rows_v[t, sl] = rows_v[t, sl] + pos_v[p, sl]
            return 0

        lax.fori_loop(0, CHUNK, add_pos, 0)

        pltpu.sync_copy(rows_v, out_hbm.at[pl.ds(base, CHUNK)])
        return carry

    lax.fori_loop(0, STEPS, step, 0)


def kernel(x, token_table, pos_table):
    xf = x.reshape(NW * STEPS, GATHERS, 128).astype(jnp.int32)
    mesh = plsc.VectorSubcoreMesh(core_axis_name="c", subcore_axis_name="s")
    run = functools.partial(
        pl.kernel,
        mesh=mesh,
        out_type=jax.ShapeDtypeStruct((TOKENS, DIM), jnp.float32),
        scratch_types=[
            pltpu.VMEM((GATHERS, 128), jnp.int32),
            pltpu.VMEM((CHUNK, DIM), jnp.float32),
            pltpu.VMEM((MAXLEN, DIM), jnp.float32),
            pltpu.SemaphoreType.DMA,
        ],
        compiler_params=pltpu.CompilerParams(use_tc_tiling_on_sc=False),
    )(_emb_kernel)
    out = run(xf, token_table, pos_table)
    return out.reshape(BATCH, MAXLEN, DIM)


# speedup vs baseline: 1.0153x; 1.0153x over previous
import functools

import jax
import jax.numpy as jnp
from jax import lax
from jax.experimental import pallas as pl
from jax.experimental.pallas import tpu as pltpu
from jax.experimental.pallas import tpu_sc as plsc

VOCAB = 1000000
DIM = 64
MAXLEN = 200
BATCH = 1024

TOKENS = BATCH * MAXLEN
NW = 32
PER_W = TOKENS // NW
BLOCKS = PER_W // 128
SUBBLK = 2
SUB = SUBBLK * 128
NSUB = BLOCKS // SUBBLK


def _emb_kernel(x_hbm, tok_hbm, pos_hbm, out_hbm,
                idx_v, idx2_v, rows_v, out_v, pos_v, sem):
    c = lax.axis_index("c")
    s = lax.axis_index("s")
    wid = s * 2 + c

    pltpu.sync_copy(pos_hbm, pos_v)
    pltpu.sync_copy(x_hbm.at[wid], idx_v)

    def halve(r, _):
        for d in range(8):
            sl = pl.ds(d * 16, 16)
            idx2_v[r, sl] = lax.shift_right_logical(idx_v[r, sl], 1)
        return 0

    lax.fori_loop(0, BLOCKS, halve, 0)

    def step(st, carry):
        blk = st * SUBBLK
        hs = []
        for j in range(SUBBLK):
            hs.append(pltpu.async_copy(
                tok_hbm.at[idx2_v.at[blk + j]],
                rows_v.at[pl.ds(j * 128, 128)],
                sem,
            ))
        for h in hs:
            h.wait()

        off = lax.rem(st * SUB, MAXLEN)

        def add_pos(t16, _):
            for j in range(SUBBLK):
                hoffv = (idx_v[blk + j, pl.ds(t16 * 16, 16)] & 1) * DIM
                for l in range(16):
                    hoff = hoffv[l]
                    row = j * 128 + t16 * 16 + l
                    p = lax.rem(off + row, MAXLEN)
                    for d in range(DIM // 16):
                        sl = pl.ds(d * 16, 16)
                        out_v[row, sl] = (rows_v[row, pl.ds(hoff + d * 16, 16)]
                                          + pos_v[p, sl])
            return 0

        lax.fori_loop(0, 8, add_pos, 0)

        pltpu.sync_copy(out_v, out_hbm.at[pl.ds(wid * PER_W + st * SUB, SUB)])
        return carry

    lax.fori_loop(0, NSUB, step, 0)


def kernel(x, token_table, pos_table):
    xw = x.reshape(NW, BLOCKS, 128).astype(jnp.int32)
    tok2 = token_table.reshape(VOCAB // 2, 2 * DIM)
    mesh = plsc.VectorSubcoreMesh(core_axis_name="c", subcore_axis_name="s")
    run = functools.partial(
        pl.kernel,
        mesh=mesh,
        out_type=jax.ShapeDtypeStruct((TOKENS, DIM), jnp.float32),
        scratch_types=[
            pltpu.VMEM((BLOCKS, 128), jnp.int32),
            pltpu.VMEM((BLOCKS, 128), jnp.int32),
            pltpu.VMEM((SUB, 2 * DIM), jnp.float32),
            pltpu.VMEM((SUB, DIM), jnp.float32),
            pltpu.VMEM((MAXLEN, DIM), jnp.float32),
            pltpu.SemaphoreType.DMA,
        ],
    )(_emb_kernel)
    out = run(xw, tok2, pos_table)
    return out.reshape(BATCH, MAXLEN, DIM)
